# baseline (speedup 1.0000x reference)
import functools

import jax
import jax.numpy as jnp
from jax import lax
from jax.experimental import pallas as pl
from jax.experimental.pallas import tpu as pltpu
from jax.experimental.pallas import tpu_sc as plsc

N = 10000
E = 320000
D = 128

_NC = 2
_NS = 16
_NW = _NC * _NS
_CHUNK = 80
_PER_W = E // _NW
_T = _PER_W // _CHUNK
_NBUF = 4


def _gather_body(ep, tp, x_hbm, ec, out, idx, ra, rb, wo, si, sga, sgb, swb):
    sid = lax.axis_index("s")
    wid = sid * _NC + lax.axis_index("c")
    per_w = ep // _NW
    w_base = wid * per_w

    def istart(t, b):
        base = w_base + t * _CHUNK
        pltpu.async_copy(ec.at[pl.ds(base, _CHUNK)], idx.at[b, 0], si.at[b])
        pltpu.async_copy(ec.at[pl.ds(ep + base, _CHUNK)], idx.at[b, 1],
                         si.at[b])

    def iwait(t, b):
        base = w_base + t * _CHUNK
        pltpu.make_async_copy(ec.at[pl.ds(base, _CHUNK)], idx.at[b, 0],
                              si.at[b]).wait()
        pltpu.make_async_copy(ec.at[pl.ds(ep + base, _CHUNK)], idx.at[b, 1],
                              si.at[b]).wait()

    def gstart(b):
        pltpu.async_copy(x_hbm.at[idx.at[b, 0]], ra.at[b], sga.at[b])
        pltpu.async_copy(x_hbm.at[idx.at[b, 1]], rb.at[b], sgb.at[b])

    def gwait(b):
        pltpu.make_async_copy(x_hbm.at[idx.at[b, 0]], ra.at[b],
                              sga.at[b]).wait()
        pltpu.make_async_copy(x_hbm.at[idx.at[b, 1]], rb.at[b],
                              sgb.at[b]).wait()

    def wbwait(t, b):
        sl = pl.ds(w_base + t * _CHUNK, _CHUNK)
        pltpu.make_async_copy(wo.at[b], out.at[sl], swb.at[b]).wait()

    def body(t, carry):
        b0 = lax.rem(t, _NBUF)
        b2 = lax.rem(t + 2, _NBUF)
        b3 = lax.rem(t + 3, _NBUF)

        @pl.when(t + 3 < tp)
        def _():
            istart(t + 3, b3)

        @pl.when(t + 2 < tp)
        def _():
            iwait(t + 2, b2)

            @pl.when(t >= 2)
            def _():
                wbwait(t - 2, b2)

            gstart(b2)

        gwait(b0)

        @plsc.parallel_loop(0, _CHUNK, unroll=4)
        def row_body(r):
            for k in range(D // 16):
                sl = pl.ds(k * 16, 16)
                wo[b0, r, sl] = ra[b0, r, sl] * rb[b0, r, sl]

        sl = pl.ds(w_base + t * _CHUNK, _CHUNK)
        pltpu.async_copy(wo.at[b0], out.at[sl], swb.at[b0])
        return carry

    for t in range(3):
        istart(t, t)
    for t in range(2):
        iwait(t, t)
        gstart(t)
    lax.fori_loop(0, tp, body, 0)
    for t in range(tp - _NBUF, tp):
        wbwait(t, t % _NBUF)


def _gather(x, ei, ep):
    tp = ep // _NW // _CHUNK
    mesh = plsc.VectorSubcoreMesh(core_axis_name="c", subcore_axis_name="s")
    f = functools.partial(
        pl.kernel,
        mesh=mesh,
        out_type=jax.ShapeDtypeStruct((ep, D), jnp.float32),
        scratch_types=[
            pltpu.VMEM((_NBUF, 2, _CHUNK), jnp.int32),
            pltpu.VMEM((_NBUF, _CHUNK, D), jnp.float32),
            pltpu.VMEM((_NBUF, _CHUNK, D), jnp.float32),
            pltpu.VMEM((_NBUF, _CHUNK, D), jnp.float32),
            pltpu.SemaphoreType.DMA((_NBUF,)),
            pltpu.SemaphoreType.DMA((_NBUF,)),
            pltpu.SemaphoreType.DMA((_NBUF,)),
            pltpu.SemaphoreType.DMA((_NBUF,)),
        ],
    )(functools.partial(_gather_body, ep, tp))
    return f(x, ei.reshape(2 * ep))


_E1 = 163840
_E2 = E - _E1
_B1 = 4096
_B2 = 2440


def _decode_pos_body(em_ref, wcat_ref, b1_ref, be1_ref, w2t_ref,
                     b2_ref, we2t_ref, be2_ref, attr_ref, pos_ref):
    h = jnp.maximum(em_ref[...], 0.0)
    a = jnp.dot(h, wcat_ref[...])
    a1 = jnp.maximum(a[:, :D] + b1_ref[...], 0.0)
    attr_ref[...] = jax.nn.sigmoid(jnp.dot(a1, w2t_ref[...]) + b2_ref[...])
    ae = jnp.maximum(a[:, D:] + be1_ref[...], 0.0)
    pos_ref[...] = jax.nn.sigmoid(jnp.dot(ae, we2t_ref[...]) + be2_ref[...])


def _decode_neg_body(em_ref, we1t_ref, be1_ref, we2t_ref, be2_ref,
                     neg_ref):
    h = jnp.maximum(em_ref[...], 0.0)
    an = jnp.maximum(jnp.dot(h, we1t_ref[...]) + be1_ref[...], 0.0)
    neg_ref[...] = jax.nn.sigmoid(jnp.dot(an, we2t_ref[...]) + be2_ref[...])


def _decode_pos(em, wcat_t, b1r, be1r, w2t8, b2r, we2t8, be2r):
    ep, b = em.shape[0], (_B1 if em.shape[0] == _E1 else _B2)
    head = pl.BlockSpec((b, 8), lambda i: (i, 0))
    head_shape = jax.ShapeDtypeStruct((ep, 8), jnp.float32)
    return pl.pallas_call(
        _decode_pos_body,
        grid=(ep // b,),
        in_specs=[
            pl.BlockSpec((b, D), lambda i: (i, 0)),
            pl.BlockSpec((D, 2 * D), lambda i: (0, 0)),
            pl.BlockSpec((1, D), lambda i: (0, 0)),
            pl.BlockSpec((1, D), lambda i: (0, 0)),
            pl.BlockSpec((D, 8), lambda i: (0, 0)),
            pl.BlockSpec((1, 8), lambda i: (0, 0)),
            pl.BlockSpec((D, 8), lambda i: (0, 0)),
            pl.BlockSpec((1, 1), lambda i: (0, 0)),
        ],
        out_specs=[head, head],
        out_shape=[head_shape, head_shape],
        compiler_params=pltpu.CompilerParams(
            dimension_semantics=("arbitrary",),
        ),
    )(em, wcat_t, b1r, be1r, w2t8, b2r, we2t8, be2r)


def _decode_neg(em, we1t, be1r, we2t8, be2r):
    ep, b = em.shape[0], (_B1 if em.shape[0] == _E1 else _B2)
    head = pl.BlockSpec((b, 8), lambda i: (i, 0))
    head_shape = jax.ShapeDtypeStruct((ep, 8), jnp.float32)
    return pl.pallas_call(
        _decode_neg_body,
        grid=(ep // b,),
        in_specs=[
            pl.BlockSpec((b, D), lambda i: (i, 0)),
            pl.BlockSpec((D, D), lambda i: (0, 0)),
            pl.BlockSpec((1, D), lambda i: (0, 0)),
            pl.BlockSpec((D, 8), lambda i: (0, 0)),
            pl.BlockSpec((1, 1), lambda i: (0, 0)),
        ],
        out_specs=[head],
        out_shape=[head_shape],
        compiler_params=pltpu.CompilerParams(
            dimension_semantics=("arbitrary",),
        ),
    )(em, we1t, be1r, we2t8, be2r)


def kernel(x, edge_index, edge_index_neg, W1, b1, W2, b2, We1, be1, We2, be2):
    em_p1 = _gather(x, edge_index[:, :_E1], _E1)
    em_p2 = _gather(x, edge_index[:, _E1:], _E2)
    em_n1 = _gather(x, edge_index_neg[:, :_E1], _E1)
    em_n2 = _gather(x, edge_index_neg[:, _E1:], _E2)

    wcat_t = jnp.concatenate([W1, We1], axis=0).T
    w2t8 = jnp.pad(W2, ((0, 1), (0, 0))).T
    b2r = jnp.pad(b2, (0, 1)).reshape(1, 8)
    we2t8 = jnp.pad(We2, ((0, 7), (0, 0))).T
    b1r = b1.reshape(1, D)
    be1r = be1.reshape(1, D)
    be2r = be2.reshape(1, 1)
    a1_8, p1_8 = _decode_pos(em_p1, wcat_t, b1r, be1r, w2t8, b2r,
                             we2t8, be2r)
    a2_8, p2_8 = _decode_pos(em_p2, wcat_t, b1r, be1r, w2t8, b2r,
                             we2t8, be2r)
    n1_8, = _decode_neg(em_n1, We1.T, be1r, we2t8, be2r)
    n2_8, = _decode_neg(em_n2, We1.T, be1r, we2t8, be2r)
    attr = jnp.concatenate([a1_8[:, :7], a2_8[:, :7]], axis=0)
    pos = jnp.concatenate([p1_8[:, 0], p2_8[:, 0]], axis=0)
    neg = jnp.concatenate([n1_8[:, 0], n2_8[:, 0]], axis=0)
    return attr, pos, neg


# speedup vs baseline: 1.1078x; 1.1078x over previous
import functools

import jax
import jax.numpy as jnp
from jax import lax
from jax.experimental import pallas as pl
from jax.experimental.pallas import tpu as pltpu
from jax.experimental.pallas import tpu_sc as plsc

N = 10000
E = 320000
D = 128

_NC = 2
_NS = 16
_NW = _NC * _NS
_CHUNK = 80
_PER_W = E // _NW
_T = _PER_W // _CHUNK
_NBUF = 4


def _gather_body(x_hbm, ec, out, idx, ra, rb, wo, si, sga, sgb, swb):
    sid = lax.axis_index("s")
    wid = sid * _NC + lax.axis_index("c")
    w_base = wid * _PER_W

    def istart(t, b):
        base = w_base + t * _CHUNK
        pltpu.async_copy(ec.at[pl.ds(base, _CHUNK)], idx.at[b, 0], si.at[b])
        pltpu.async_copy(ec.at[pl.ds(E + base, _CHUNK)], idx.at[b, 1],
                         si.at[b])

    def iwait(t, b):
        base = w_base + t * _CHUNK
        pltpu.make_async_copy(ec.at[pl.ds(base, _CHUNK)], idx.at[b, 0],
                              si.at[b]).wait()
        pltpu.make_async_copy(ec.at[pl.ds(E + base, _CHUNK)], idx.at[b, 1],
                              si.at[b]).wait()

    def gstart(b):
        pltpu.async_copy(x_hbm.at[idx.at[b, 0]], ra.at[b], sga.at[b])
        pltpu.async_copy(x_hbm.at[idx.at[b, 1]], rb.at[b], sgb.at[b])

    def gwait(b):
        pltpu.make_async_copy(x_hbm.at[idx.at[b, 0]], ra.at[b],
                              sga.at[b]).wait()
        pltpu.make_async_copy(x_hbm.at[idx.at[b, 1]], rb.at[b],
                              sgb.at[b]).wait()

    def wbwait(t, b):
        sl = pl.ds(w_base + t * _CHUNK, _CHUNK)
        pltpu.make_async_copy(wo.at[b], out.at[sl], swb.at[b]).wait()

    def body(t, carry):
        b0 = lax.rem(t, _NBUF)
        b2 = lax.rem(t + 2, _NBUF)
        b3 = lax.rem(t + 3, _NBUF)

        @pl.when(t + 3 < _T)
        def _():
            istart(t + 3, b3)

        @pl.when(t + 2 < _T)
        def _():
            iwait(t + 2, b2)

            @pl.when(t >= 2)
            def _():
                wbwait(t - 2, b2)

            gstart(b2)

        gwait(b0)

        @plsc.parallel_loop(0, _CHUNK, unroll=4)
        def row_body(r):
            for k in range(D // 16):
                sl = pl.ds(k * 16, 16)
                wo[b0, r, sl] = ra[b0, r, sl] * rb[b0, r, sl]

        sl = pl.ds(w_base + t * _CHUNK, _CHUNK)
        pltpu.async_copy(wo.at[b0], out.at[sl], swb.at[b0])
        return carry

    for t in range(3):
        istart(t, t)
    for t in range(2):
        iwait(t, t)
        gstart(t)
    lax.fori_loop(0, _T, body, 0)
    for t in range(_T - _NBUF, _T):
        wbwait(t, t % _NBUF)


def _gather(x, ei):
    mesh = plsc.VectorSubcoreMesh(core_axis_name="c", subcore_axis_name="s")
    f = functools.partial(
        pl.kernel,
        mesh=mesh,
        out_type=jax.ShapeDtypeStruct((E, D), jnp.float32),
        scratch_types=[
            pltpu.VMEM((_NBUF, 2, _CHUNK), jnp.int32),
            pltpu.VMEM((_NBUF, _CHUNK, D), jnp.float32),
            pltpu.VMEM((_NBUF, _CHUNK, D), jnp.float32),
            pltpu.VMEM((_NBUF, _CHUNK, D), jnp.float32),
            pltpu.SemaphoreType.DMA((_NBUF,)),
            pltpu.SemaphoreType.DMA((_NBUF,)),
            pltpu.SemaphoreType.DMA((_NBUF,)),
            pltpu.SemaphoreType.DMA((_NBUF,)),
        ],
    )(_gather_body)
    return f(x, ei.reshape(2 * E))


_B = 4000
_G = E // _B

_EWISE = pl.BlockSpec((_B, D), lambda i: (i, 0))
_HEAD = pl.BlockSpec((_B, 8), lambda i: (i, 0))
_HEAD_SHAPE = jax.ShapeDtypeStruct((E, 8), jnp.float32)


def _decode_pos_body(em_ref, wcat_ref, b1_ref, be1_ref, w2t_ref,
                     b2_ref, we2t_ref, be2_ref, attr_ref, pos_ref):
    h = jnp.maximum(em_ref[...], 0.0)
    a = jnp.dot(h, wcat_ref[...])
    a1 = jnp.maximum(a[:, :D] + b1_ref[...], 0.0)
    attr_ref[...] = jax.nn.sigmoid(jnp.dot(a1, w2t_ref[...]) + b2_ref[...])
    ae = jnp.maximum(a[:, D:] + be1_ref[...], 0.0)
    pos_ref[...] = jax.nn.sigmoid(jnp.dot(ae, we2t_ref[...]) + be2_ref[...])


def _decode_neg_body(em_ref, we1t_ref, be1_ref, we2t_ref, be2_ref,
                     neg_ref):
    h = jnp.maximum(em_ref[...], 0.0)
    an = jnp.maximum(jnp.dot(h, we1t_ref[...]) + be1_ref[...], 0.0)
    neg_ref[...] = jax.nn.sigmoid(jnp.dot(an, we2t_ref[...]) + be2_ref[...])


def _decode_pos(em, wcat_t, b1r, be1r, w2t8, b2r, we2t8, be2r):
    return pl.pallas_call(
        _decode_pos_body,
        grid=(_G,),
        in_specs=[
            _EWISE,
            pl.BlockSpec((D, 2 * D), lambda i: (0, 0)),
            pl.BlockSpec((1, D), lambda i: (0, 0)),
            pl.BlockSpec((1, D), lambda i: (0, 0)),
            pl.BlockSpec((D, 8), lambda i: (0, 0)),
            pl.BlockSpec((1, 8), lambda i: (0, 0)),
            pl.BlockSpec((D, 8), lambda i: (0, 0)),
            pl.BlockSpec((1, 1), lambda i: (0, 0)),
        ],
        out_specs=[_HEAD, _HEAD],
        out_shape=[_HEAD_SHAPE, _HEAD_SHAPE],
        compiler_params=pltpu.CompilerParams(
            dimension_semantics=("arbitrary",),
        ),
    )(em, wcat_t, b1r, be1r, w2t8, b2r, we2t8, be2r)


def _decode_neg(em, we1t, be1r, we2t8, be2r):
    return pl.pallas_call(
        _decode_neg_body,
        grid=(_G,),
        in_specs=[
            _EWISE,
            pl.BlockSpec((D, D), lambda i: (0, 0)),
            pl.BlockSpec((1, D), lambda i: (0, 0)),
            pl.BlockSpec((D, 8), lambda i: (0, 0)),
            pl.BlockSpec((1, 1), lambda i: (0, 0)),
        ],
        out_specs=[_HEAD],
        out_shape=[_HEAD_SHAPE],
        compiler_params=pltpu.CompilerParams(
            dimension_semantics=("arbitrary",),
        ),
    )(em, we1t, be1r, we2t8, be2r)


def kernel(x, edge_index, edge_index_neg, W1, b1, W2, b2, We1, be1, We2, be2):
    em_pos = _gather(x, edge_index)
    em_neg = _gather(x, edge_index_neg)

    wcat_t = jnp.concatenate([W1, We1], axis=0).T
    w2t8 = jnp.pad(W2, ((0, 1), (0, 0))).T
    b2r = jnp.pad(b2, (0, 1)).reshape(1, 8)
    we2t8 = jnp.pad(We2, ((0, 7), (0, 0))).T
    b1r = b1.reshape(1, D)
    be1r = be1.reshape(1, D)
    be2r = be2.reshape(1, 1)
    attr8, pos8 = _decode_pos(em_pos, wcat_t, b1r, be1r, w2t8, b2r,
                              we2t8, be2r)
    neg8, = _decode_neg(em_neg, We1.T, be1r, we2t8, be2r)
    return attr8[:, :7], pos8[:, 0], neg8[:, 0]
